# initial kernel scaffold (unmeasured)
import jax
import jax.numpy as jnp
from jax import lax
from jax.experimental import pallas as pl
from jax.experimental.pallas import tpu as pltpu


def kernel(
    x,
):
    def body(*refs):
        pass

    out_shape = jax.ShapeDtypeStruct(..., jnp.float32)
    return pl.pallas_call(body, out_shape=out_shape)(...)


# baseline (device time: 30530 ns/iter reference)
import jax
import jax.numpy as jnp
from jax import lax
from jax.experimental import pallas as pl
from jax.experimental.pallas import tpu as pltpu

N_DEV = 16


def kernel(x):
    m, n = x.shape
    chunk = m // N_DEV

    def body(x_ref, out_ref, stage_ref, rs_ref, ag_ref,
             rs_send_sems, rs_recv_sems, ag_send_sems, ag_recv_sems):
        my = lax.axis_index("i")

        barrier_sem = pltpu.get_barrier_semaphore()
        for o in range(1, N_DEV):
            pl.semaphore_signal(
                barrier_sem, inc=1,
                device_id=((my + o) % N_DEV,),
                device_id_type=pl.DeviceIdType.MESH,
            )
        pl.semaphore_wait(barrier_sem, N_DEV - 1)

        stage_ref[...] = x_ref[...].reshape(N_DEV, chunk, n).astype(jnp.bfloat16)

        rs_sends = []
        for o in range(1, N_DEV):
            dest = (my + o) % N_DEV
            rdma = pltpu.make_async_remote_copy(
                src_ref=stage_ref.at[dest],
                dst_ref=rs_ref.at[my],
                send_sem=rs_send_sems.at[o],
                recv_sem=rs_recv_sems.at[my],
                device_id=(dest,),
                device_id_type=pl.DeviceIdType.MESH,
            )
            rdma.start()
            rs_sends.append(rdma)

        rs_ref[pl.ds(my, 1)] = stage_ref[pl.ds(my, 1)]

        for o in range(1, N_DEV):
            src = (my + o) % N_DEV
            pltpu.make_async_remote_copy(
                src_ref=stage_ref.at[0],
                dst_ref=rs_ref.at[src],
                send_sem=rs_send_sems.at[0],
                recv_sem=rs_recv_sems.at[src],
                device_id=(my,),
                device_id_type=pl.DeviceIdType.MESH,
            ).wait_recv()

        reduced = jnp.sum(rs_ref[...].astype(jnp.float32), axis=0)
        ag_ref[pl.ds(my, 1)] = reduced[None].astype(jnp.bfloat16)

        ag_sends = []
        for o in range(1, N_DEV):
            dest = (my + o) % N_DEV
            rdma = pltpu.make_async_remote_copy(
                src_ref=ag_ref.at[my],
                dst_ref=ag_ref.at[my],
                send_sem=ag_send_sems.at[o],
                recv_sem=ag_recv_sems.at[my],
                device_id=(dest,),
                device_id_type=pl.DeviceIdType.MESH,
            )
            rdma.start()
            ag_sends.append(rdma)

        for o in range(1, N_DEV):
            src = (my + o) % N_DEV
            pltpu.make_async_remote_copy(
                src_ref=ag_ref.at[0],
                dst_ref=ag_ref.at[src],
                send_sem=ag_send_sems.at[0],
                recv_sem=ag_recv_sems.at[src],
                device_id=(my,),
                device_id_type=pl.DeviceIdType.MESH,
            ).wait_recv()

        out_ref[...] = ag_ref[...].astype(jnp.float32).reshape(m, n)

        for rdma in rs_sends + ag_sends:
            rdma.wait_send()

    return pl.pallas_call(
        body,
        out_shape=jax.ShapeDtypeStruct((m, n), jnp.float32),
        in_specs=[pl.BlockSpec(memory_space=pltpu.VMEM)],
        out_specs=pl.BlockSpec(memory_space=pltpu.VMEM),
        scratch_shapes=[
            pltpu.VMEM((N_DEV, chunk, n), jnp.bfloat16),
            pltpu.VMEM((N_DEV, chunk, n), jnp.bfloat16),
            pltpu.VMEM((N_DEV, chunk, n), jnp.bfloat16),
            pltpu.SemaphoreType.DMA((N_DEV,)),
            pltpu.SemaphoreType.DMA((N_DEV,)),
            pltpu.SemaphoreType.DMA((N_DEV,)),
            pltpu.SemaphoreType.DMA((N_DEV,)),
        ],
        compiler_params=pltpu.CompilerParams(collective_id=0),
    )(x)
